# baseline (speedup 1.0000x reference)
import functools

import jax
import jax.numpy as jnp
from jax import lax
from jax.experimental import pallas as pl
from jax.experimental.pallas import tpu as pltpu
from jax.experimental.pallas import tpu_sc as plsc

CHUNK = 80


@functools.lru_cache(maxsize=None)
def _make_gather(N, D):
    info = plsc.get_sparse_core_info()
    NC, NS = info.num_cores, info.num_subcores
    NW = NC * NS
    assert N % (NW * CHUNK) == 0
    n_per_w = N // NW
    n_chunks = n_per_w // CHUNK
    assert n_chunks % 2 == 0 and n_chunks >= 4
    mesh = plsc.VectorSubcoreMesh(core_axis_name="c", subcore_axis_name="s")

    @functools.partial(
        pl.kernel,
        mesh=mesh,
        out_type=jax.ShapeDtypeStruct((N, D), jnp.float32),
        compiler_params=pltpu.CompilerParams(use_tc_tiling_on_sc=False),
        scratch_types=[
            pltpu.VMEM((n_per_w,), jnp.int32),
            pltpu.VMEM((CHUNK, D), jnp.float32),
            pltpu.VMEM((CHUNK, D), jnp.float32),
            pltpu.SemaphoreType.DMA,
            pltpu.SemaphoreType.DMA,
            pltpu.SemaphoreType.DMA,
            pltpu.SemaphoreType.DMA,
        ],
    )
    def gather_kernel(x_hbm, table_hbm, out_hbm, idx_v, buf0, buf1,
                      sg0, sg1, so0, so1):
        wid = lax.axis_index("s") * NC + lax.axis_index("c")
        base = wid * n_per_w
        bufs = (buf0, buf1)
        sgs = (sg0, sg1)
        sos = (so0, so1)

        def start_gather(c, b):
            pltpu.async_copy(
                table_hbm.at[idx_v.at[pl.ds(c * CHUNK, CHUNK)]],
                bufs[b], sgs[b])

        def wait_gather(b):
            pltpu.make_async_copy(
                table_hbm.at[idx_v.at[pl.ds(0, CHUNK)]], bufs[b],
                sgs[b]).wait()

        def start_out(c, b):
            pltpu.async_copy(
                bufs[b], out_hbm.at[pl.ds(base + c * CHUNK, CHUNK)], sos[b])

        def wait_out(c, b):
            pltpu.make_async_copy(
                bufs[b], out_hbm.at[pl.ds(base + c * CHUNK, CHUNK)],
                sos[b]).wait()

        pltpu.sync_copy(x_hbm.at[pl.ds(base, n_per_w)], idx_v)

        start_gather(0, 0)
        wait_gather(0)
        start_out(0, 0)
        start_gather(1, 1)

        def body(i, carry):
            c = 1 + 2 * i
            for b, cc in ((1, c), (0, c + 1)):
                wait_gather(b)
                start_out(cc, b)
                wait_out(cc - 1, b ^ 1)
                start_gather(cc + 1, b ^ 1)
            return carry

        lax.fori_loop(0, (n_chunks - 2) // 2, body, 0)

        wait_gather(1)
        start_out(n_chunks - 1, 1)
        wait_out(n_chunks - 2, 0)
        wait_out(n_chunks - 1, 1)

    return gather_kernel


def _kernel_impl(x, word_emb):
    B, S = x.shape
    D = word_emb.shape[1]
    flat = _make_gather(B * S, D)(x.reshape(B * S).astype(jnp.int32), word_emb)
    return flat.reshape(B, S, D)


from jax.experimental.layout import Format, Layout

kernel = jax.jit(
    _kernel_impl,
    out_shardings=Format(
        Layout(major_to_minor=(0, 1, 2), tiling=()),
        jax.sharding.SingleDeviceSharding(jax.devices()[0]),
    ),
)


# speedup vs baseline: 1.0623x; 1.0623x over previous
import functools

import jax
import jax.numpy as jnp
from jax import lax
from jax.experimental import pallas as pl
from jax.experimental.pallas import tpu as pltpu
from jax.experimental.pallas import tpu_sc as plsc

CHUNK = 80


@functools.lru_cache(maxsize=None)
def _make_gather(N, P):
    info = plsc.get_sparse_core_info()
    NC, NS = info.num_cores, info.num_subcores
    NW = NC * NS
    assert N % (NW * CHUNK) == 0
    n_per_w = N // NW
    i_per_w = 4 * n_per_w
    IC = 4 * CHUNK
    n_chunks = n_per_w // CHUNK
    assert n_chunks % 2 == 0 and n_chunks >= 4
    mesh = plsc.VectorSubcoreMesh(core_axis_name="c", subcore_axis_name="s")

    @functools.partial(
        pl.kernel,
        mesh=mesh,
        out_type=jax.ShapeDtypeStruct((4 * N, 128), jnp.float32),
        compiler_params=pltpu.CompilerParams(use_tc_tiling_on_sc=False),
        scratch_types=[
            pltpu.VMEM((i_per_w,), jnp.int32),
            pltpu.VMEM((IC, 128), jnp.float32),
            pltpu.VMEM((IC, 128), jnp.float32),
            pltpu.SemaphoreType.DMA,
            pltpu.SemaphoreType.DMA,
            pltpu.SemaphoreType.DMA,
            pltpu.SemaphoreType.DMA,
        ],
    )
    def gather_kernel(px_hbm, table_hbm, out_hbm, idx_v, buf0, buf1,
                      sg0, sg1, so0, so1):
        wid = lax.axis_index("s") * NC + lax.axis_index("c")
        base = wid * i_per_w
        bufs = (buf0, buf1)
        sgs = (sg0, sg1)
        sos = (so0, so1)

        def start_gather(c, b):
            pltpu.async_copy(
                table_hbm.at[idx_v.at[pl.ds(c * IC, IC)]], bufs[b], sgs[b])

        def wait_gather(b):
            pltpu.make_async_copy(
                table_hbm.at[idx_v.at[pl.ds(0, IC)]], bufs[b], sgs[b]).wait()

        def start_out(c, b):
            pltpu.async_copy(
                bufs[b], out_hbm.at[pl.ds(base + c * IC, IC)], sos[b])

        def wait_out(c, b):
            pltpu.make_async_copy(
                bufs[b], out_hbm.at[pl.ds(base + c * IC, IC)], sos[b]).wait()

        pltpu.sync_copy(px_hbm.at[pl.ds(base, i_per_w)], idx_v)

        start_gather(0, 0)
        wait_gather(0)
        start_out(0, 0)
        start_gather(1, 1)

        def body(i, carry):
            c = 1 + 2 * i
            for b, cc in ((1, c), (0, c + 1)):
                wait_gather(b)
                start_out(cc, b)
                wait_out(cc - 1, b ^ 1)
                start_gather(cc + 1, b ^ 1)
            return carry

        lax.fori_loop(0, (n_chunks - 2) // 2, body, 0)

        wait_gather(1)
        start_out(n_chunks - 1, 1)
        wait_out(n_chunks - 2, 0)
        wait_out(n_chunks - 1, 1)

    return gather_kernel


def _kernel_impl(x, word_emb):
    B, S = x.shape
    V, D = word_emb.shape
    table4 = (word_emb.reshape(V // 8, 8, D // 128, 128)
              .transpose(0, 2, 1, 3).reshape(V * (D // 128), 128))
    v = x.reshape(B * S).astype(jnp.int32)
    px = ((v >> 3) * 32 + (v & 7))[:, None] + 8 * jnp.arange(4, dtype=jnp.int32)
    flat = _make_gather(B * S, V * (D // 128))(px.reshape(-1), table4)
    return flat.reshape(B, S, D)


from jax.experimental.layout import Format, Layout

kernel = jax.jit(
    _kernel_impl,
    out_shardings=Format(
        Layout(major_to_minor=(0, 1, 2), tiling=()),
        jax.sharding.SingleDeviceSharding(jax.devices()[0]),
    ),
)
